# baseline (speedup 1.0000x reference)
import jax
import jax.numpy as jnp
from jax.experimental import pallas as pl
from jax.experimental.pallas import tpu as pltpu

_B = 1024
_S = 4096
_K = 4
_E = 16
_BM = 64
_BS = 2048


def _body(x_ref, e_ref, p_ref, o_ref):
    p = p_ref[...]
    e = e_ref[...]
    for b in range(_BM):
        y = jnp.dot(e, x_ref[b], preferred_element_type=jnp.float32)
        o_ref[b] = y + p


def kernel(inputs, embedding, pos_table):
    xt = jnp.transpose(inputs, (0, 2, 1))
    pt = jnp.transpose(pos_table, (1, 0))
    et = jnp.transpose(embedding, (1, 0))
    out_t = pl.pallas_call(
        _body,
        grid=(_B // _BM, _S // _BS),
        in_specs=[
            pl.BlockSpec((_BM, _K, _BS), lambda i, j: (i, 0, j)),
            pl.BlockSpec((_E, _K), lambda i, j: (0, 0)),
            pl.BlockSpec((_E, _BS), lambda i, j: (0, j)),
        ],
        out_specs=pl.BlockSpec((_BM, _E, _BS), lambda i, j: (i, 0, j)),
        out_shape=jax.ShapeDtypeStruct((_B, _E, _S), jnp.float32),
        compiler_params=pltpu.CompilerParams(
            dimension_semantics=("parallel", "parallel"),
        ),
    )(xt, et, pt)
    return jnp.transpose(out_t, (0, 2, 1))


# speedup vs baseline: 1.0455x; 1.0455x over previous
import jax
import jax.numpy as jnp
from jax.experimental import pallas as pl
from jax.experimental.pallas import tpu as pltpu

_B = 1024
_S = 4096
_K = 4
_E = 16
_BM = 64
_BS = 4096


def _body(x_ref, e_ref, p_ref, o_ref):
    p = p_ref[...]
    e = e_ref[...]
    for b in range(_BM):
        y = jnp.dot(e, x_ref[b], preferred_element_type=jnp.float32)
        o_ref[b] = y + p


def kernel(inputs, embedding, pos_table):
    xt = jnp.transpose(inputs, (0, 2, 1))
    pt = jnp.transpose(pos_table, (1, 0))
    et = jnp.transpose(embedding, (1, 0))
    out_t = pl.pallas_call(
        _body,
        grid=(_B // _BM, _S // _BS),
        in_specs=[
            pl.BlockSpec((_BM, _K, _BS), lambda i, j: (i, 0, j)),
            pl.BlockSpec((_E, _K), lambda i, j: (0, 0)),
            pl.BlockSpec((_E, _BS), lambda i, j: (0, j)),
        ],
        out_specs=pl.BlockSpec((_BM, _E, _BS), lambda i, j: (i, 0, j)),
        out_shape=jax.ShapeDtypeStruct((_B, _E, _S), jnp.float32),
        compiler_params=pltpu.CompilerParams(
            dimension_semantics=("arbitrary", "arbitrary"),
        ),
    )(xt, et, pt)
    return jnp.transpose(out_t, (0, 2, 1))
